# baseline (speedup 1.0000x reference)
import functools
import jax
import jax.numpy as jnp
from jax import lax
from jax.experimental import pallas as pl
from jax.experimental.pallas import tpu as pltpu
from jax.experimental.pallas import tpu_sc as plsc

NEG = jnp.finfo(jnp.float32).min

_N = 8192
_NW = 32
_SC_ROWS = 4096
_RPW = _SC_ROWS // _NW
_RC = 4
_NCH = _RPW // _RC
_LANES = 16
_KV = _N // _LANES


def _sc_sum_body(adj_hbm, f1_hbm, out_hbm, f1_v, buf0, buf1, out_v, acc_buf,
                 sem0, sem1):
    wid = lax.axis_index("s") * 2 + lax.axis_index("c")
    base = wid * _RPW
    pltpu.sync_copy(f1_hbm, f1_v)
    bufs = (buf0, buf1)
    sems = (sem0, sem1)
    lanes = lax.iota(jnp.int32, _LANES)

    pltpu.async_copy(adj_hbm.at[pl.ds(base, _RC)], buf0, sem0)

    def group_body(g, _):
        for cc in range(_LANES // _RC):
            c = g * (_LANES // _RC) + cc
            p = cc % 2
            buf = bufs[p]
            row0 = base + c * _RC
            pltpu.make_async_copy(adj_hbm.at[pl.ds(row0, _RC)], buf, sems[p]).wait()

            @pl.when(c + 1 < _NCH)
            def _prefetch():
                pltpu.async_copy(
                    adj_hbm.at[pl.ds(row0 + _RC, _RC)], bufs[1 - p], sems[1 - p])

            def kbody(k, accs):
                off = k * _LANES
                f = f1_v[pl.ds(off, _LANES)]
                return tuple(
                    accs[r] + jnp.where(buf[r, pl.ds(off, _LANES)] != 0, f, 0.0)
                    for r in range(_RC))

            accs = lax.fori_loop(
                0, _KV, kbody, tuple(jnp.zeros((_LANES,), jnp.float32)
                                     for _ in range(_RC)))
            for r in range(_RC):
                acc_buf[cc * _RC + r, :] = accs[r]
        res = jnp.zeros((_LANES,), jnp.float32)
        for t in range(_LANES):
            col = jnp.full((_LANES,), t, jnp.int32)
            res = res + plsc.load_gather(acc_buf, [lanes, col])
        out_v[pl.ds(g * _LANES, _LANES)] = res
        return 0

    lax.fori_loop(0, _RPW // _LANES, group_body, 0)
    pltpu.sync_copy(out_v, out_hbm.at[pl.ds(base, _RPW)])


def _sc_scal2_top(adj_1to2, f1_row):
    mesh = plsc.VectorSubcoreMesh(core_axis_name="c", subcore_axis_name="s")
    return pl.kernel(
        _sc_sum_body,
        out_type=jax.ShapeDtypeStruct((_SC_ROWS,), jnp.float32),
        mesh=mesh,
        compiler_params=pltpu.CompilerParams(needs_layout_passes=False),
        scratch_types=[
            pltpu.VMEM((_N,), jnp.float32),
            pltpu.VMEM((_RC, _N), jnp.int32),
            pltpu.VMEM((_RC, _N), jnp.int32),
            pltpu.VMEM((_RPW,), jnp.float32),
            pltpu.VMEM((_LANES, _LANES), jnp.float32),
            pltpu.SemaphoreType.DMA,
            pltpu.SemaphoreType.DMA,
        ],
    )(adj_1to2, f1_row)


def _tc_max_body(adj21, f2, x1, w1s, w1n, out1, m_acc, h_acc, *, n_col_blocks):
    c = pl.program_id(1)
    a21 = adj21[...]
    vals = jnp.where(a21 != 0, f2[...], NEG)
    m = jnp.max(vals, axis=1, keepdims=True)
    h = jnp.max(a21, axis=1, keepdims=True)

    @pl.when(c == 0)
    def _init():
        m_acc[...] = m
        h_acc[...] = h

    @pl.when(c > 0)
    def _accum():
        m_acc[...] = jnp.maximum(m_acc[...], m)
        h_acc[...] = jnp.maximum(h_acc[...], h)

    @pl.when(c == n_col_blocks - 1)
    def _finalize():
        scal1 = jnp.where(h_acc[...] > 0, m_acc[...], 0.0)
        wsum1 = jnp.sum(w1n[...], axis=1)
        o1 = jnp.dot(x1[...], w1s[...].T, preferred_element_type=jnp.float32)
        out1[...] = jnp.maximum(o1 + scal1 * wsum1[None, :], 0.0)


def _tc_sum_body(adj12, f1, out_s, s_acc, *, n_col_blocks):
    c = pl.program_id(1)
    s = jnp.sum(jnp.where(adj12[...] != 0, f1[...], 0.0), axis=1, keepdims=True)

    @pl.when(c == 0)
    def _init():
        s_acc[...] = s

    @pl.when(c > 0)
    def _accum():
        s_acc[...] = s_acc[...] + s

    @pl.when(c == n_col_blocks - 1)
    def _finalize():
        out_s[...] = s_acc[...]


def _tc_out2_body(scal2, x2, w2s, w2n, out2):
    wsum2 = jnp.sum(w2n[...], axis=1)
    o2 = jnp.dot(x2[...], w2s[...].T, preferred_element_type=jnp.float32)
    out2[...] = jnp.maximum(o2 + scal2[...] * wsum2[None, :], 0.0)


def kernel(node_feats1, node_feats2, adj_1to2, adj_2to1,
           W1_self, W1_neigh, W2_self, W2_neigh):
    n1, d_in = node_feats1.shape
    n2, _ = node_feats2.shape
    d_out = W1_self.shape[0]

    f1_row = node_feats1[:, 0]
    f2_row = node_feats2[:, 0].reshape(1, n2)

    scal2_top = _sc_scal2_top(adj_1to2, f1_row)

    br = 256
    bc = 2048
    nc = n1 // bc

    nb = n2 - _SC_ROWS
    scal2_bot = pl.pallas_call(
        functools.partial(_tc_sum_body, n_col_blocks=nc),
        grid=(nb // br, nc),
        in_specs=[
            pl.BlockSpec((br, bc), lambda r, c: (r, c)),
            pl.BlockSpec((1, bc), lambda r, c: (0, c)),
        ],
        out_specs=pl.BlockSpec((br, 1), lambda r, c: (r, 0)),
        out_shape=jax.ShapeDtypeStruct((nb, 1), jnp.float32),
        scratch_shapes=[pltpu.VMEM((br, 1), jnp.float32)],
        compiler_params=pltpu.CompilerParams(
            dimension_semantics=("parallel", "arbitrary"),
        ),
    )(lax.slice_in_dim(adj_1to2, _SC_ROWS, n2, axis=0), f1_row.reshape(1, n1))

    out1 = pl.pallas_call(
        functools.partial(_tc_max_body, n_col_blocks=nc),
        grid=(n1 // br, nc),
        in_specs=[
            pl.BlockSpec((br, bc), lambda r, c: (r, c)),
            pl.BlockSpec((1, bc), lambda r, c: (0, c)),
            pl.BlockSpec((br, d_in), lambda r, c: (r, 0)),
            pl.BlockSpec((d_out, d_in), lambda r, c: (0, 0)),
            pl.BlockSpec((d_out, d_in), lambda r, c: (0, 0)),
        ],
        out_specs=pl.BlockSpec((br, d_out), lambda r, c: (r, 0)),
        out_shape=jax.ShapeDtypeStruct((n1, d_out), jnp.float32),
        scratch_shapes=[
            pltpu.VMEM((br, 1), jnp.float32),
            pltpu.VMEM((br, 1), jnp.int32),
        ],
        compiler_params=pltpu.CompilerParams(
            dimension_semantics=("parallel", "arbitrary"),
        ),
    )(adj_2to1, f2_row, node_feats1, W1_self, W1_neigh)

    scal2 = jnp.concatenate([scal2_top.reshape(_SC_ROWS, 1), scal2_bot], axis=0)

    out2 = pl.pallas_call(
        _tc_out2_body,
        grid=(n2 // br,),
        in_specs=[
            pl.BlockSpec((br, 1), lambda r: (r, 0)),
            pl.BlockSpec((br, d_in), lambda r: (r, 0)),
            pl.BlockSpec((d_out, d_in), lambda r: (0, 0)),
            pl.BlockSpec((d_out, d_in), lambda r: (0, 0)),
        ],
        out_specs=pl.BlockSpec((br, d_out), lambda r: (r, 0)),
        out_shape=jax.ShapeDtypeStruct((n2, d_out), jnp.float32),
        compiler_params=pltpu.CompilerParams(
            dimension_semantics=("arbitrary",),
        ),
    )(scal2, node_feats2, W2_self, W2_neigh)

    return out1, out2


# speedup vs baseline: 1.3590x; 1.3590x over previous
import functools
import jax
import jax.numpy as jnp
from jax import lax
from jax.experimental import pallas as pl
from jax.experimental.pallas import tpu as pltpu
from jax.experimental.pallas import tpu_sc as plsc

NEG = jnp.finfo(jnp.float32).min

_N = 8192
_NW = 32
_SC_ROWS = 4096
_RPW = _SC_ROWS // _NW
_RC = 4
_NCH = _RPW // _RC
_LANES = 16
_KV = _N // _LANES


def _sc_sum_body(adj_hbm, f1_hbm, out_hbm, f1_v, buf0, buf1, out_v, acc_buf,
                 sem0, sem1):
    wid = lax.axis_index("s") * 2 + lax.axis_index("c")
    base = wid * _RPW
    pltpu.sync_copy(f1_hbm, f1_v)
    bufs = (buf0, buf1)
    sems = (sem0, sem1)
    lanes = lax.iota(jnp.int32, _LANES)

    pltpu.async_copy(adj_hbm.at[pl.ds(base, _RC)], buf0, sem0)

    def group_body(g, _):
        for cc in range(_LANES // _RC):
            c = g * (_LANES // _RC) + cc
            p = cc % 2
            buf = bufs[p]
            row0 = base + c * _RC
            pltpu.make_async_copy(adj_hbm.at[pl.ds(row0, _RC)], buf, sems[p]).wait()

            @pl.when(c + 1 < _NCH)
            def _prefetch():
                pltpu.async_copy(
                    adj_hbm.at[pl.ds(row0 + _RC, _RC)], bufs[1 - p], sems[1 - p])

            def kbody(k, accs):
                off = k * _LANES
                f = f1_v[pl.ds(off, _LANES)]
                return tuple(
                    accs[r] + jnp.where(buf[r, pl.ds(off, _LANES)] != 0, f, 0.0)
                    for r in range(_RC))

            accs = lax.fori_loop(
                0, _KV, kbody, tuple(jnp.zeros((_LANES,), jnp.float32)
                                     for _ in range(_RC)))
            for r in range(_RC):
                acc_buf[cc * _RC + r, :] = accs[r]
        res = jnp.zeros((_LANES,), jnp.float32)
        for t in range(_LANES):
            col = jnp.full((_LANES,), t, jnp.int32)
            res = res + plsc.load_gather(acc_buf, [lanes, col])
        out_v[pl.ds(g * _LANES, _LANES)] = res
        return 0

    lax.fori_loop(0, _RPW // _LANES, group_body, 0)
    pltpu.sync_copy(out_v, out_hbm.at[pl.ds(base, _RPW)])


def _sc_scal2_top(adj_1to2, f1_row):
    mesh = plsc.VectorSubcoreMesh(core_axis_name="c", subcore_axis_name="s")
    return pl.kernel(
        _sc_sum_body,
        out_type=jax.ShapeDtypeStruct((_SC_ROWS,), jnp.float32),
        mesh=mesh,
        compiler_params=pltpu.CompilerParams(needs_layout_passes=False),
        scratch_types=[
            pltpu.VMEM((_N,), jnp.float32),
            pltpu.VMEM((_RC, _N), jnp.int32),
            pltpu.VMEM((_RC, _N), jnp.int32),
            pltpu.VMEM((_RPW,), jnp.float32),
            pltpu.VMEM((_LANES, _LANES), jnp.float32),
            pltpu.SemaphoreType.DMA,
            pltpu.SemaphoreType.DMA,
        ],
    )(adj_1to2, f1_row)


def _tc_max_body(adj21, f2, x1, w1s, w1n, out1, m_acc, h_acc, *, n_col_blocks):
    c = pl.program_id(1)
    a21 = adj21[...]
    vals = jnp.where(a21 != 0, f2[...], NEG)
    m = jnp.max(vals, axis=1, keepdims=True)
    h = jnp.max(a21, axis=1, keepdims=True)

    @pl.when(c == 0)
    def _init():
        m_acc[...] = m
        h_acc[...] = h

    @pl.when(c > 0)
    def _accum():
        m_acc[...] = jnp.maximum(m_acc[...], m)
        h_acc[...] = jnp.maximum(h_acc[...], h)

    @pl.when(c == n_col_blocks - 1)
    def _finalize():
        scal1 = jnp.where(h_acc[...] > 0, m_acc[...], 0.0)
        wsum1 = jnp.sum(w1n[...], axis=1)
        o1 = jnp.dot(x1[...], w1s[...].T, preferred_element_type=jnp.float32)
        out1[...] = jnp.maximum(o1 + scal1 * wsum1[None, :], 0.0)


def _tc_sum_body(adj12, f1, out_s, s_acc, *, n_col_blocks):
    c = pl.program_id(1)
    s = jnp.sum(jnp.where(adj12[...] != 0, f1[...], 0.0), axis=1, keepdims=True)

    @pl.when(c == 0)
    def _init():
        s_acc[...] = s

    @pl.when(c > 0)
    def _accum():
        s_acc[...] = s_acc[...] + s

    @pl.when(c == n_col_blocks - 1)
    def _finalize():
        out_s[...] = s_acc[...]


def _tc_out2_body(scal2, x2, w2s, w2n, out2):
    wsum2 = jnp.sum(w2n[...], axis=1)
    o2 = jnp.dot(x2[...], w2s[...].T, preferred_element_type=jnp.float32)
    out2[...] = jnp.maximum(o2 + scal2[...] * wsum2[None, :], 0.0)


def kernel(node_feats1, node_feats2, adj_1to2, adj_2to1,
           W1_self, W1_neigh, W2_self, W2_neigh):
    n1, d_in = node_feats1.shape
    n2, _ = node_feats2.shape
    d_out = W1_self.shape[0]

    f1_row = node_feats1[:, 0]
    f2_row = node_feats2[:, 0].reshape(1, n2)

    scal2_top = _sc_scal2_top(adj_1to2, f1_row)

    br = 256
    bc = 2048
    nc = n1 // bc

    nb = n2 - _SC_ROWS
    scal2_bot = pl.pallas_call(
        functools.partial(_tc_sum_body, n_col_blocks=nc),
        grid=(nb // br, nc),
        in_specs=[
            pl.BlockSpec((br, bc), lambda r, c: (r + _SC_ROWS // 256, c)),
            pl.BlockSpec((1, bc), lambda r, c: (0, c)),
        ],
        out_specs=pl.BlockSpec((br, 1), lambda r, c: (r, 0)),
        out_shape=jax.ShapeDtypeStruct((nb, 1), jnp.float32),
        scratch_shapes=[pltpu.VMEM((br, 1), jnp.float32)],
        compiler_params=pltpu.CompilerParams(
            dimension_semantics=("parallel", "arbitrary"),
        ),
    )(adj_1to2, f1_row.reshape(1, n1))

    out1 = pl.pallas_call(
        functools.partial(_tc_max_body, n_col_blocks=nc),
        grid=(n1 // br, nc),
        in_specs=[
            pl.BlockSpec((br, bc), lambda r, c: (r, c)),
            pl.BlockSpec((1, bc), lambda r, c: (0, c)),
            pl.BlockSpec((br, d_in), lambda r, c: (r, 0)),
            pl.BlockSpec((d_out, d_in), lambda r, c: (0, 0)),
            pl.BlockSpec((d_out, d_in), lambda r, c: (0, 0)),
        ],
        out_specs=pl.BlockSpec((br, d_out), lambda r, c: (r, 0)),
        out_shape=jax.ShapeDtypeStruct((n1, d_out), jnp.float32),
        scratch_shapes=[
            pltpu.VMEM((br, 1), jnp.float32),
            pltpu.VMEM((br, 1), jnp.int32),
        ],
        compiler_params=pltpu.CompilerParams(
            dimension_semantics=("parallel", "arbitrary"),
        ),
    )(adj_2to1, f2_row, node_feats1, W1_self, W1_neigh)

    scal2 = jnp.concatenate([scal2_top.reshape(_SC_ROWS, 1), scal2_bot], axis=0)

    out2 = pl.pallas_call(
        _tc_out2_body,
        grid=(n2 // br,),
        in_specs=[
            pl.BlockSpec((br, 1), lambda r: (r, 0)),
            pl.BlockSpec((br, d_in), lambda r: (r, 0)),
            pl.BlockSpec((d_out, d_in), lambda r: (0, 0)),
            pl.BlockSpec((d_out, d_in), lambda r: (0, 0)),
        ],
        out_specs=pl.BlockSpec((br, d_out), lambda r: (r, 0)),
        out_shape=jax.ShapeDtypeStruct((n2, d_out), jnp.float32),
        compiler_params=pltpu.CompilerParams(
            dimension_semantics=("arbitrary",),
        ),
    )(scal2, node_feats2, W2_self, W2_neigh)

    return out1, out2


# speedup vs baseline: 1.6122x; 1.1863x over previous
import functools
import jax
import jax.numpy as jnp
from jax import lax
from jax.experimental import pallas as pl
from jax.experimental.pallas import tpu as pltpu
from jax.experimental.pallas import tpu_sc as plsc

NEG = jnp.finfo(jnp.float32).min

_N = 8192
_NW = 32
_SC_ROWS = 4096
_RPW = _SC_ROWS // _NW
_RC = 4
_NCH = _RPW // _RC
_LANES = 16
_KV = _N // _LANES

_BR = 256
_BC = 2048
_BR2 = (_N - _SC_ROWS) // (_N // _BR)


def _sc_sum_body(adj_hbm, f1_hbm, out_hbm, f1_v, buf0, buf1, out_v, acc_buf,
                 sem0, sem1):
    wid = lax.axis_index("s") * 2 + lax.axis_index("c")
    base = wid * _RPW
    pltpu.sync_copy(f1_hbm, f1_v)
    bufs = (buf0, buf1)
    sems = (sem0, sem1)
    lanes = lax.iota(jnp.int32, _LANES)

    pltpu.async_copy(adj_hbm.at[pl.ds(base, _RC)], buf0, sem0)

    def group_body(g, _):
        for cc in range(_LANES // _RC):
            c = g * (_LANES // _RC) + cc
            p = cc % 2
            buf = bufs[p]
            row0 = base + c * _RC
            pltpu.make_async_copy(adj_hbm.at[pl.ds(row0, _RC)], buf, sems[p]).wait()

            @pl.when(c + 1 < _NCH)
            def _prefetch():
                pltpu.async_copy(
                    adj_hbm.at[pl.ds(row0 + _RC, _RC)], bufs[1 - p], sems[1 - p])

            def kbody(k, accs):
                off = k * _LANES
                f = f1_v[pl.ds(off, _LANES)]
                return tuple(
                    accs[r] + jnp.where(buf[r, pl.ds(off, _LANES)] != 0, f, 0.0)
                    for r in range(_RC))

            accs = lax.fori_loop(
                0, _KV, kbody, tuple(jnp.zeros((_LANES,), jnp.float32)
                                     for _ in range(_RC)))
            for r in range(_RC):
                acc_buf[cc * _RC + r, :] = accs[r]
        res = jnp.zeros((_LANES,), jnp.float32)
        for t in range(_LANES):
            col = jnp.full((_LANES,), t, jnp.int32)
            res = res + plsc.load_gather(acc_buf, [lanes, col])
        out_v[pl.ds(g * _LANES, _LANES)] = res
        return 0

    lax.fori_loop(0, _RPW // _LANES, group_body, 0)
    pltpu.sync_copy(out_v, out_hbm.at[pl.ds(base, _RPW)])


def _sc_scal2_top(adj_1to2, f1_row):
    mesh = plsc.VectorSubcoreMesh(core_axis_name="c", subcore_axis_name="s")
    return pl.kernel(
        _sc_sum_body,
        out_type=jax.ShapeDtypeStruct((_SC_ROWS,), jnp.float32),
        mesh=mesh,
        compiler_params=pltpu.CompilerParams(needs_layout_passes=False),
        scratch_types=[
            pltpu.VMEM((_N,), jnp.float32),
            pltpu.VMEM((_RC, _N), jnp.int32),
            pltpu.VMEM((_RC, _N), jnp.int32),
            pltpu.VMEM((_RPW,), jnp.float32),
            pltpu.VMEM((_LANES, _LANES), jnp.float32),
            pltpu.SemaphoreType.DMA,
            pltpu.SemaphoreType.DMA,
        ],
    )(adj_1to2, f1_row)


def _tc_main_body(adj21, adj12, f2, f1, x1, x2b, w1s, w1n, w2s, w2n,
                  out1, out2b, m_acc, h_acc, s_acc, *, n_col_blocks):
    c = pl.program_id(1)

    a21 = adj21[...]
    vals = jnp.where(a21 != 0, f2[...], NEG)
    m = jnp.max(vals, axis=1, keepdims=True)
    h = jnp.max(a21, axis=1, keepdims=True)
    s = jnp.sum(jnp.where(adj12[...] != 0, f1[...], 0.0), axis=1, keepdims=True)

    @pl.when(c == 0)
    def _init():
        m_acc[...] = m
        h_acc[...] = h
        s_acc[...] = s

    @pl.when(c > 0)
    def _accum():
        m_acc[...] = jnp.maximum(m_acc[...], m)
        h_acc[...] = jnp.maximum(h_acc[...], h)
        s_acc[...] = s_acc[...] + s

    @pl.when(c == n_col_blocks - 1)
    def _finalize():
        scal1 = jnp.where(h_acc[...] > 0, m_acc[...], 0.0)
        wsum1 = jnp.sum(w1n[...], axis=1)
        o1 = jnp.dot(x1[...], w1s[...].T, preferred_element_type=jnp.float32)
        out1[...] = jnp.maximum(o1 + scal1 * wsum1[None, :], 0.0)
        wsum2 = jnp.sum(w2n[...], axis=1)
        o2 = jnp.dot(x2b[...], w2s[...].T, preferred_element_type=jnp.float32)
        out2b[...] = jnp.maximum(o2 + s_acc[...] * wsum2[None, :], 0.0)


def _tc_out2_body(scal2, x2, w2s, w2n, out2):
    wsum2 = jnp.sum(w2n[...], axis=1)
    o2 = jnp.dot(x2[...], w2s[...].T, preferred_element_type=jnp.float32)
    out2[...] = jnp.maximum(o2 + scal2[...] * wsum2[None, :], 0.0)


def kernel(node_feats1, node_feats2, adj_1to2, adj_2to1,
           W1_self, W1_neigh, W2_self, W2_neigh):
    n1, d_in = node_feats1.shape
    n2, _ = node_feats2.shape
    d_out = W1_self.shape[0]

    f1_row = node_feats1[:, 0]
    f2_row = node_feats2[:, 0].reshape(1, n2)

    scal2_top = _sc_scal2_top(adj_1to2, f1_row)

    nr = n1 // _BR
    nc = n2 // _BC
    rb2 = _SC_ROWS // _BR2

    out1, out2b = pl.pallas_call(
        functools.partial(_tc_main_body, n_col_blocks=nc),
        grid=(nr, nc),
        in_specs=[
            pl.BlockSpec((_BR, _BC), lambda r, c: (r, c)),
            pl.BlockSpec((_BR2, _BC), lambda r, c: (rb2 + r, c)),
            pl.BlockSpec((1, _BC), lambda r, c: (0, c)),
            pl.BlockSpec((1, _BC), lambda r, c: (0, c)),
            pl.BlockSpec((_BR, d_in), lambda r, c: (r, 0)),
            pl.BlockSpec((_BR2, d_in), lambda r, c: (rb2 + r, 0)),
            pl.BlockSpec((d_out, d_in), lambda r, c: (0, 0)),
            pl.BlockSpec((d_out, d_in), lambda r, c: (0, 0)),
            pl.BlockSpec((d_out, d_in), lambda r, c: (0, 0)),
            pl.BlockSpec((d_out, d_in), lambda r, c: (0, 0)),
        ],
        out_specs=[
            pl.BlockSpec((_BR, d_out), lambda r, c: (r, 0)),
            pl.BlockSpec((_BR2, d_out), lambda r, c: (r, 0)),
        ],
        out_shape=[
            jax.ShapeDtypeStruct((n1, d_out), jnp.float32),
            jax.ShapeDtypeStruct((n2 - _SC_ROWS, d_out), jnp.float32),
        ],
        scratch_shapes=[
            pltpu.VMEM((_BR, 1), jnp.float32),
            pltpu.VMEM((_BR, 1), jnp.int32),
            pltpu.VMEM((_BR2, 1), jnp.float32),
        ],
        compiler_params=pltpu.CompilerParams(
            dimension_semantics=("parallel", "arbitrary"),
        ),
    )(adj_2to1, adj_1to2, f2_row, f1_row.reshape(1, n1), node_feats1,
      node_feats2, W1_self, W1_neigh, W2_self, W2_neigh)

    br2 = 512
    out2t = pl.pallas_call(
        _tc_out2_body,
        grid=(_SC_ROWS // br2,),
        in_specs=[
            pl.BlockSpec((br2, 1), lambda r: (r, 0)),
            pl.BlockSpec((br2, d_in), lambda r: (r, 0)),
            pl.BlockSpec((d_out, d_in), lambda r: (0, 0)),
            pl.BlockSpec((d_out, d_in), lambda r: (0, 0)),
        ],
        out_specs=pl.BlockSpec((br2, d_out), lambda r: (r, 0)),
        out_shape=jax.ShapeDtypeStruct((_SC_ROWS, d_out), jnp.float32),
        compiler_params=pltpu.CompilerParams(
            dimension_semantics=("arbitrary",),
        ),
    )(scal2_top.reshape(_SC_ROWS, 1), node_feats2, W2_self, W2_neigh)

    out2 = jnp.concatenate([out2t, out2b], axis=0)
    return out1, out2


# speedup vs baseline: 1.6132x; 1.0007x over previous
import functools
import jax
import jax.numpy as jnp
from jax import lax
from jax.experimental import pallas as pl
from jax.experimental.pallas import tpu as pltpu
from jax.experimental.pallas import tpu_sc as plsc

NEG = jnp.finfo(jnp.float32).min

_N = 8192
_NW = 32
_SC_ROWS = 4096
_RPW = _SC_ROWS // _NW
_RC = 4
_NCH = _RPW // _RC
_LANES = 16
_KV = _N // _LANES
_KU = 4

_BR = 256
_BC = 2048
_BR2 = (_N - _SC_ROWS) // (_N // _BR)


def _sc_sum_body(adj_hbm, f1_hbm, out_hbm, f1_v, buf0, buf1, out_v, acc_buf,
                 sem0, sem1):
    wid = lax.axis_index("s") * 2 + lax.axis_index("c")
    base = wid * _RPW
    pltpu.sync_copy(f1_hbm, f1_v)
    bufs = (buf0, buf1)
    sems = (sem0, sem1)
    lanes = lax.iota(jnp.int32, _LANES)

    pltpu.async_copy(adj_hbm.at[pl.ds(base, _RC)], buf0, sem0)

    def group_body(g, _):
        for cc in range(_LANES // _RC):
            c = g * (_LANES // _RC) + cc
            p = cc % 2
            buf = bufs[p]
            row0 = base + c * _RC
            pltpu.make_async_copy(adj_hbm.at[pl.ds(row0, _RC)], buf, sems[p]).wait()

            @pl.when(c + 1 < _NCH)
            def _prefetch():
                pltpu.async_copy(
                    adj_hbm.at[pl.ds(row0 + _RC, _RC)], bufs[1 - p], sems[1 - p])

            def kbody(k, accs):
                accs = list(accs)
                for u in range(_KU):
                    off = (k * _KU + u) * _LANES
                    f = f1_v[pl.ds(off, _LANES)]
                    for r in range(_RC):
                        a = buf[r, pl.ds(off, _LANES)].astype(jnp.float32)
                        accs[r] = accs[r] + a * f
                return tuple(accs)

            accs = lax.fori_loop(
                0, _KV // _KU, kbody, tuple(jnp.zeros((_LANES,), jnp.float32)
                                            for _ in range(_RC)))
            for r in range(_RC):
                acc_buf[cc * _RC + r, :] = accs[r]
        res = jnp.zeros((_LANES,), jnp.float32)
        for t in range(_LANES):
            col = jnp.full((_LANES,), t, jnp.int32)
            res = res + plsc.load_gather(acc_buf, [lanes, col])
        out_v[pl.ds(g * _LANES, _LANES)] = res
        return 0

    lax.fori_loop(0, _RPW // _LANES, group_body, 0)
    pltpu.sync_copy(out_v, out_hbm.at[pl.ds(base, _RPW)])


def _sc_scal2_top(adj_1to2, f1_row):
    mesh = plsc.VectorSubcoreMesh(core_axis_name="c", subcore_axis_name="s")
    return pl.kernel(
        _sc_sum_body,
        out_type=jax.ShapeDtypeStruct((_SC_ROWS,), jnp.float32),
        mesh=mesh,
        compiler_params=pltpu.CompilerParams(needs_layout_passes=False),
        scratch_types=[
            pltpu.VMEM((_N,), jnp.float32),
            pltpu.VMEM((_RC, _N), jnp.int32),
            pltpu.VMEM((_RC, _N), jnp.int32),
            pltpu.VMEM((_RPW,), jnp.float32),
            pltpu.VMEM((_LANES, _LANES), jnp.float32),
            pltpu.SemaphoreType.DMA,
            pltpu.SemaphoreType.DMA,
        ],
    )(adj_1to2, f1_row)


def _tc_main_body(adj21, adj12, f2, f1, x1, x2b, w1s, w1n, w2s, w2n,
                  out1, out2b, m_acc, h_acc, s_acc, *, n_col_blocks):
    c = pl.program_id(1)

    a21 = adj21[...]
    vals = jnp.where(a21 != 0, f2[...], NEG)
    m = jnp.max(vals, axis=1, keepdims=True)
    h = jnp.max(a21, axis=1, keepdims=True)
    s = jnp.sum(jnp.where(adj12[...] != 0, f1[...], 0.0), axis=1, keepdims=True)

    @pl.when(c == 0)
    def _init():
        m_acc[...] = m
        h_acc[...] = h
        s_acc[...] = s

    @pl.when(c > 0)
    def _accum():
        m_acc[...] = jnp.maximum(m_acc[...], m)
        h_acc[...] = jnp.maximum(h_acc[...], h)
        s_acc[...] = s_acc[...] + s

    @pl.when(c == n_col_blocks - 1)
    def _finalize():
        scal1 = jnp.where(h_acc[...] > 0, m_acc[...], 0.0)
        wsum1 = jnp.sum(w1n[...], axis=1)
        o1 = jnp.dot(x1[...], w1s[...].T, preferred_element_type=jnp.float32)
        out1[...] = jnp.maximum(o1 + scal1 * wsum1[None, :], 0.0)
        wsum2 = jnp.sum(w2n[...], axis=1)
        o2 = jnp.dot(x2b[...], w2s[...].T, preferred_element_type=jnp.float32)
        out2b[...] = jnp.maximum(o2 + s_acc[...] * wsum2[None, :], 0.0)


def _tc_out2_body(scal2, x2, w2s, w2n, out2):
    wsum2 = jnp.sum(w2n[...], axis=1)
    o2 = jnp.dot(x2[...], w2s[...].T, preferred_element_type=jnp.float32)
    out2[...] = jnp.maximum(o2 + scal2[...] * wsum2[None, :], 0.0)


def kernel(node_feats1, node_feats2, adj_1to2, adj_2to1,
           W1_self, W1_neigh, W2_self, W2_neigh):
    n1, d_in = node_feats1.shape
    n2, _ = node_feats2.shape
    d_out = W1_self.shape[0]

    f1_row = node_feats1[:, 0]
    f2_row = node_feats2[:, 0].reshape(1, n2)

    scal2_top = _sc_scal2_top(adj_1to2, f1_row)

    nr = n1 // _BR
    nc = n2 // _BC
    rb2 = _SC_ROWS // _BR2

    out1, out2b = pl.pallas_call(
        functools.partial(_tc_main_body, n_col_blocks=nc),
        grid=(nr, nc),
        in_specs=[
            pl.BlockSpec((_BR, _BC), lambda r, c: (r, c)),
            pl.BlockSpec((_BR2, _BC), lambda r, c: (rb2 + r, c)),
            pl.BlockSpec((1, _BC), lambda r, c: (0, c)),
            pl.BlockSpec((1, _BC), lambda r, c: (0, c)),
            pl.BlockSpec((_BR, d_in), lambda r, c: (r, 0)),
            pl.BlockSpec((_BR2, d_in), lambda r, c: (rb2 + r, 0)),
            pl.BlockSpec((d_out, d_in), lambda r, c: (0, 0)),
            pl.BlockSpec((d_out, d_in), lambda r, c: (0, 0)),
            pl.BlockSpec((d_out, d_in), lambda r, c: (0, 0)),
            pl.BlockSpec((d_out, d_in), lambda r, c: (0, 0)),
        ],
        out_specs=[
            pl.BlockSpec((_BR, d_out), lambda r, c: (r, 0)),
            pl.BlockSpec((_BR2, d_out), lambda r, c: (r, 0)),
        ],
        out_shape=[
            jax.ShapeDtypeStruct((n1, d_out), jnp.float32),
            jax.ShapeDtypeStruct((n2 - _SC_ROWS, d_out), jnp.float32),
        ],
        scratch_shapes=[
            pltpu.VMEM((_BR, 1), jnp.float32),
            pltpu.VMEM((_BR, 1), jnp.int32),
            pltpu.VMEM((_BR2, 1), jnp.float32),
        ],
        compiler_params=pltpu.CompilerParams(
            dimension_semantics=("parallel", "arbitrary"),
        ),
    )(adj_2to1, adj_1to2, f2_row, f1_row.reshape(1, n1), node_feats1,
      node_feats2, W1_self, W1_neigh, W2_self, W2_neigh)

    br2 = 512
    out2t = pl.pallas_call(
        _tc_out2_body,
        grid=(_SC_ROWS // br2,),
        in_specs=[
            pl.BlockSpec((br2, 1), lambda r: (r, 0)),
            pl.BlockSpec((br2, d_in), lambda r: (r, 0)),
            pl.BlockSpec((d_out, d_in), lambda r: (0, 0)),
            pl.BlockSpec((d_out, d_in), lambda r: (0, 0)),
        ],
        out_specs=pl.BlockSpec((br2, d_out), lambda r: (r, 0)),
        out_shape=jax.ShapeDtypeStruct((_SC_ROWS, d_out), jnp.float32),
        compiler_params=pltpu.CompilerParams(
            dimension_semantics=("arbitrary",),
        ),
    )(scal2_top.reshape(_SC_ROWS, 1), node_feats2, W2_self, W2_neigh)

    out2 = jnp.concatenate([out2t, out2b], axis=0)
    return out1, out2


# speedup vs baseline: 2.2136x; 1.3722x over previous
import functools
import jax
import jax.numpy as jnp
from jax.experimental import pallas as pl
from jax.experimental.pallas import tpu as pltpu

NEG = jnp.finfo(jnp.float32).min


def _body(adj21, adj12, f2, f1, x1, x2, w1s, w1n, w2s, w2n,
          out1, out2, m_acc, h_acc, s_acc, *, n_col_blocks):
    c = pl.program_id(1)

    a21 = adj21[...]
    a12 = adj12[...]
    vals = jnp.where(a21 != 0, f2[...], NEG)
    m = jnp.max(vals, axis=1, keepdims=True)
    h = jnp.max(a21, axis=1, keepdims=True)
    s = jnp.sum(jnp.where(a12 != 0, f1[...], 0.0), axis=1, keepdims=True)

    @pl.when(c == 0)
    def _init():
        m_acc[...] = m
        h_acc[...] = h
        s_acc[...] = s

    @pl.when(c > 0)
    def _accum():
        m_acc[...] = jnp.maximum(m_acc[...], m)
        h_acc[...] = jnp.maximum(h_acc[...], h)
        s_acc[...] = s_acc[...] + s

    @pl.when(c == n_col_blocks - 1)
    def _finalize():
        scal1 = jnp.where(h_acc[...] > 0, m_acc[...], 0.0)
        scal2 = s_acc[...]
        wsum1 = jnp.sum(w1n[...], axis=1)
        wsum2 = jnp.sum(w2n[...], axis=1)
        o1 = jnp.dot(x1[...], w1s[...].T, preferred_element_type=jnp.float32)
        o2 = jnp.dot(x2[...], w2s[...].T, preferred_element_type=jnp.float32)
        out1[...] = jnp.maximum(o1 + scal1 * wsum1[None, :], 0.0)
        out2[...] = jnp.maximum(o2 + scal2 * wsum2[None, :], 0.0)


def kernel(node_feats1, node_feats2, adj_1to2, adj_2to1,
           W1_self, W1_neigh, W2_self, W2_neigh):
    n1, d_in = node_feats1.shape
    n2, _ = node_feats2.shape
    d_out = W1_self.shape[0]

    br = 256
    bc = 4096
    nr = n1 // br
    nc = n2 // bc

    f2_row = node_feats2[:, 0].reshape(1, n2)
    f1_row = node_feats1[:, 0].reshape(1, n1)

    grid = (nr, nc)
    out1, out2 = pl.pallas_call(
        functools.partial(_body, n_col_blocks=nc),
        grid=grid,
        in_specs=[
            pl.BlockSpec((br, bc), lambda r, c: (r, c)),
            pl.BlockSpec((br, bc), lambda r, c: (r, c)),
            pl.BlockSpec((1, bc), lambda r, c: (0, c)),
            pl.BlockSpec((1, bc), lambda r, c: (0, c)),
            pl.BlockSpec((br, d_in), lambda r, c: (r, 0)),
            pl.BlockSpec((br, d_in), lambda r, c: (r, 0)),
            pl.BlockSpec((d_out, d_in), lambda r, c: (0, 0)),
            pl.BlockSpec((d_out, d_in), lambda r, c: (0, 0)),
            pl.BlockSpec((d_out, d_in), lambda r, c: (0, 0)),
            pl.BlockSpec((d_out, d_in), lambda r, c: (0, 0)),
        ],
        out_specs=[
            pl.BlockSpec((br, d_out), lambda r, c: (r, 0)),
            pl.BlockSpec((br, d_out), lambda r, c: (r, 0)),
        ],
        out_shape=[
            jax.ShapeDtypeStruct((n1, d_out), jnp.float32),
            jax.ShapeDtypeStruct((n2, d_out), jnp.float32),
        ],
        scratch_shapes=[
            pltpu.VMEM((br, 1), jnp.float32),
            pltpu.VMEM((br, 1), jnp.int32),
            pltpu.VMEM((br, 1), jnp.float32),
        ],
        compiler_params=pltpu.CompilerParams(
            dimension_semantics=("parallel", "arbitrary"),
        ),
    )(adj_2to1, adj_1to2, f2_row, f1_row, node_feats1, node_feats2,
      W1_self, W1_neigh, W2_self, W2_neigh)
    return out1, out2


# speedup vs baseline: 2.2393x; 1.0116x over previous
import functools
import jax
import jax.numpy as jnp
from jax.experimental import pallas as pl
from jax.experimental.pallas import tpu as pltpu

NEG = jnp.finfo(jnp.float32).min


def _body(adj21, adj12, f2, f1, x1, x2, w1s, w1n, w2s, w2n,
          out1, out2, m_acc, h_acc, s_acc, *, n_col_blocks):
    c = pl.program_id(1)

    a21 = adj21[...]
    a12 = adj12[...]
    vals = jnp.where(a21 != 0, f2[...], NEG)
    m = jnp.max(vals, axis=1, keepdims=True)
    h = jnp.max(a21, axis=1, keepdims=True)
    s = jnp.sum(jnp.where(a12 != 0, f1[...], 0.0), axis=1, keepdims=True)

    @pl.when(c == 0)
    def _init():
        m_acc[...] = m
        h_acc[...] = h
        s_acc[...] = s

    @pl.when(c > 0)
    def _accum():
        m_acc[...] = jnp.maximum(m_acc[...], m)
        h_acc[...] = jnp.maximum(h_acc[...], h)
        s_acc[...] = s_acc[...] + s

    @pl.when(c == n_col_blocks - 1)
    def _finalize():
        scal1 = jnp.where(h_acc[...] > 0, m_acc[...], 0.0)
        scal2 = s_acc[...]
        wsum1 = jnp.sum(w1n[...], axis=1)
        wsum2 = jnp.sum(w2n[...], axis=1)
        o1 = jnp.dot(x1[...], w1s[...].T, preferred_element_type=jnp.float32)
        o2 = jnp.dot(x2[...], w2s[...].T, preferred_element_type=jnp.float32)
        out1[...] = jnp.maximum(o1 + scal1 * wsum1[None, :], 0.0)
        out2[...] = jnp.maximum(o2 + scal2 * wsum2[None, :], 0.0)


def kernel(node_feats1, node_feats2, adj_1to2, adj_2to1,
           W1_self, W1_neigh, W2_self, W2_neigh):
    n1, d_in = node_feats1.shape
    n2, _ = node_feats2.shape
    d_out = W1_self.shape[0]

    br = 256
    bc = 8192
    nr = n1 // br
    nc = n2 // bc

    f2_row = node_feats2[:, 0].reshape(1, n2)
    f1_row = node_feats1[:, 0].reshape(1, n1)

    grid = (nr, nc)
    out1, out2 = pl.pallas_call(
        functools.partial(_body, n_col_blocks=nc),
        grid=grid,
        in_specs=[
            pl.BlockSpec((br, bc), lambda r, c: (r, c)),
            pl.BlockSpec((br, bc), lambda r, c: (r, c)),
            pl.BlockSpec((1, bc), lambda r, c: (0, c)),
            pl.BlockSpec((1, bc), lambda r, c: (0, c)),
            pl.BlockSpec((br, d_in), lambda r, c: (r, 0)),
            pl.BlockSpec((br, d_in), lambda r, c: (r, 0)),
            pl.BlockSpec((d_out, d_in), lambda r, c: (0, 0)),
            pl.BlockSpec((d_out, d_in), lambda r, c: (0, 0)),
            pl.BlockSpec((d_out, d_in), lambda r, c: (0, 0)),
            pl.BlockSpec((d_out, d_in), lambda r, c: (0, 0)),
        ],
        out_specs=[
            pl.BlockSpec((br, d_out), lambda r, c: (r, 0)),
            pl.BlockSpec((br, d_out), lambda r, c: (r, 0)),
        ],
        out_shape=[
            jax.ShapeDtypeStruct((n1, d_out), jnp.float32),
            jax.ShapeDtypeStruct((n2, d_out), jnp.float32),
        ],
        scratch_shapes=[
            pltpu.VMEM((br, 1), jnp.float32),
            pltpu.VMEM((br, 1), jnp.int32),
            pltpu.VMEM((br, 1), jnp.float32),
        ],
        compiler_params=pltpu.CompilerParams(
            dimension_semantics=("parallel", "arbitrary"),
        ),
    )(adj_2to1, adj_1to2, f2_row, f1_row, node_feats1, node_feats2,
      W1_self, W1_neigh, W2_self, W2_neigh)
    return out1, out2


# speedup vs baseline: 2.2464x; 1.0032x over previous
import jax
import jax.numpy as jnp
from jax.experimental import pallas as pl
from jax.experimental.pallas import tpu as pltpu

NEG = jnp.finfo(jnp.float32).min


def _body(adj21, adj12, f2, f1, x1, x2, w1s, w1n, w2s, w2n, out1, out2):
    a21 = adj21[...]
    a12 = adj12[...]
    vals = jnp.where(a21 != 0, f2[...], NEG)
    m = jnp.max(vals, axis=1, keepdims=True)
    h = jnp.max(a21, axis=1, keepdims=True)
    s = jnp.sum(jnp.where(a12 != 0, f1[...], 0.0), axis=1, keepdims=True)

    scal1 = jnp.where(h > 0, m, 0.0)
    wsum1 = jnp.sum(w1n[...], axis=1)
    wsum2 = jnp.sum(w2n[...], axis=1)
    o1 = jnp.dot(x1[...], w1s[...].T, preferred_element_type=jnp.float32)
    o2 = jnp.dot(x2[...], w2s[...].T, preferred_element_type=jnp.float32)
    out1[...] = jnp.maximum(o1 + scal1 * wsum1[None, :], 0.0)
    out2[...] = jnp.maximum(o2 + s * wsum2[None, :], 0.0)


def kernel(node_feats1, node_feats2, adj_1to2, adj_2to1,
           W1_self, W1_neigh, W2_self, W2_neigh):
    n1, d_in = node_feats1.shape
    n2, _ = node_feats2.shape
    d_out = W1_self.shape[0]

    br = 256
    nr = n1 // br

    f2_row = node_feats2[:, 0].reshape(1, n2)
    f1_row = node_feats1[:, 0].reshape(1, n1)

    out1, out2 = pl.pallas_call(
        _body,
        grid=(nr,),
        in_specs=[
            pl.BlockSpec((br, n2), lambda r: (r, 0)),
            pl.BlockSpec((br, n1), lambda r: (r, 0)),
            pl.BlockSpec((1, n2), lambda r: (0, 0)),
            pl.BlockSpec((1, n1), lambda r: (0, 0)),
            pl.BlockSpec((br, d_in), lambda r: (r, 0)),
            pl.BlockSpec((br, d_in), lambda r: (r, 0)),
            pl.BlockSpec((d_out, d_in), lambda r: (0, 0)),
            pl.BlockSpec((d_out, d_in), lambda r: (0, 0)),
            pl.BlockSpec((d_out, d_in), lambda r: (0, 0)),
            pl.BlockSpec((d_out, d_in), lambda r: (0, 0)),
        ],
        out_specs=[
            pl.BlockSpec((br, d_out), lambda r: (r, 0)),
            pl.BlockSpec((br, d_out), lambda r: (r, 0)),
        ],
        out_shape=[
            jax.ShapeDtypeStruct((n1, d_out), jnp.float32),
            jax.ShapeDtypeStruct((n2, d_out), jnp.float32),
        ],
        compiler_params=pltpu.CompilerParams(
            dimension_semantics=("parallel",),
        ),
    )(adj_2to1, adj_1to2, f2_row, f1_row, node_feats1, node_feats2,
      W1_self, W1_neigh, W2_self, W2_neigh)
    return out1, out2
